# baseline (speedup 1.0000x reference)
import functools

import jax
import jax.numpy as jnp
from jax import lax
from jax.experimental import pallas as pl
from jax.experimental.pallas import tpu as pltpu
from jax.experimental.pallas import tpu_sc as plsc

N = 10000
E = 320000
D = 128
TH = 0.1
LN_EPS = 1e-5

NPAD = 10240
NCORE = 2
NSUB = 16
NTILE = NCORE * NSUB
EPT = E // NTILE
CH = 80
SEG = NPAD // NSUB

_f32 = jnp.float32

_mesh = plsc.VectorSubcoreMesh(core_axis_name="c", subcore_axis_name="s")


def _pre_body(h_ref, w_ref, hn_ref, hw_ref):
    h = h_ref[...]
    hw_ref[...] = jnp.dot(h, w_ref[...], preferred_element_type=_f32)
    nr = jnp.sqrt(jnp.sum(h * h, axis=1, keepdims=True))
    hn_ref[...] = h / jnp.maximum(nr, 1e-8)


def _dense_pre(h, W):
    return pl.pallas_call(
        _pre_body,
        grid=(8,),
        in_specs=[
            pl.BlockSpec((NPAD // 8, D), lambda i: (i, 0)),
            pl.BlockSpec((D, D), lambda i: (0, 0)),
        ],
        out_specs=[
            pl.BlockSpec((NPAD // 8, D), lambda i: (i, 0)),
            pl.BlockSpec((NPAD // 8, D), lambda i: (i, 0)),
        ],
        out_shape=[
            jax.ShapeDtypeStruct((NPAD, D), _f32),
            jax.ShapeDtypeStruct((NPAD, D), _f32),
        ],
    )(h, W)


def _norm_body(degp_ref, cntp_ref, dis_ref, sv_ref, *, cadd):
    deg = degp_ref[0] + degp_ref[1]
    dis_ref[...] = jnp.where(deg > 0.0, lax.rsqrt(deg), 0.0)
    cnt = cntp_ref[0] + cntp_ref[1]
    sv_ref[...] = jnp.exp(1.0 / (cnt + cadd))


def _norm_tc(degp, cntp, cadd):
    degp2 = degp.reshape(2, NPAD // 128, 128)
    cntp2 = cntp.reshape(2, NPAD // 128, 128)
    dis, sv = pl.pallas_call(
        functools.partial(_norm_body, cadd=cadd),
        out_shape=[
            jax.ShapeDtypeStruct((NPAD // 128, 128), _f32),
            jax.ShapeDtypeStruct((NPAD // 128, 128), _f32),
        ],
    )(degp2, cntp2)
    return dis.reshape(NPAD), sv.reshape(NPAD)


def _post_body(outp_ref, sv_ref, hw_ref, b_ref, g_ref, be_ref, h_ref):
    o = outp_ref[0] + outp_ref[1] + sv_ref[...] * hw_ref[...] + b_ref[...]
    m = jnp.mean(o, axis=1, keepdims=True)
    v = jnp.mean((o - m) ** 2, axis=1, keepdims=True)
    h_ref[...] = jnp.maximum(
        (o - m) / jnp.sqrt(v + LN_EPS) * g_ref[...] + be_ref[...], 0.0
    )


def _final_body(outp_ref, sv_ref, hw_ref, b_ref, h_ref):
    o = outp_ref[0] + outp_ref[1] + sv_ref[...] * hw_ref[...] + b_ref[...]
    z = o - jnp.max(o, axis=1, keepdims=True)
    h_ref[...] = z - jnp.log(jnp.sum(jnp.exp(z), axis=1, keepdims=True))


def _dense_post(outp, sv, hw, b, g, be):
    blk = NPAD // 8
    return pl.pallas_call(
        _post_body,
        grid=(8,),
        in_specs=[
            pl.BlockSpec((2, blk, D), lambda i: (0, i, 0)),
            pl.BlockSpec((blk, 1), lambda i: (i, 0)),
            pl.BlockSpec((blk, D), lambda i: (i, 0)),
            pl.BlockSpec((1, D), lambda i: (0, 0)),
            pl.BlockSpec((1, D), lambda i: (0, 0)),
            pl.BlockSpec((1, D), lambda i: (0, 0)),
        ],
        out_specs=pl.BlockSpec((blk, D), lambda i: (i, 0)),
        out_shape=jax.ShapeDtypeStruct((NPAD, D), _f32),
    )(outp, sv[:, None], hw, b[None, :], g[None, :], be[None, :])


def _dense_final(outp, sv, hw, b):
    blk = NPAD // 8
    return pl.pallas_call(
        _final_body,
        grid=(8,),
        in_specs=[
            pl.BlockSpec((2, blk, D), lambda i: (0, i, 0)),
            pl.BlockSpec((blk, 1), lambda i: (i, 0)),
            pl.BlockSpec((blk, D), lambda i: (i, 0)),
            pl.BlockSpec((1, D), lambda i: (0, 0)),
        ],
        out_specs=pl.BlockSpec((blk, D), lambda i: (i, 0)),
        out_shape=jax.ShapeDtypeStruct((NPAD, D), _f32),
    )(outp, sv[:, None], hw, b[None, :])


CAP = EPT + 2 * CH
DUM = NPAD - 1


def _pass_a_body(hn, crow_in, ccol_in, counts_in,
                 crow_o, ccol_o, csims_o, counts_o, deg_o, cnt_o,
                 idxr, idxc, sidxr, sidxc, rr, rc, keepv, skv, zbuf, cbuf,
                 crow_v, ccol_v, csims_v,
                 degs, cnts, semi, semx):
    cid = lax.axis_index("c")
    sid = lax.axis_index("s")
    wid = cid * NSUB + sid

    zz = jnp.zeros((16,), _f32)

    def zb(i, _):
        zbuf[pl.ds(i * 16, 16)] = zz
        return 0

    lax.fori_loop(0, SEG // 16, zb, 0)
    pltpu.sync_copy(zbuf, degs.at[pl.ds(sid * SEG, SEG)])
    pltpu.sync_copy(zbuf, cnts.at[pl.ds(sid * SEG, SEG)])
    plsc.subcore_barrier()

    pltpu.sync_copy(counts_in.at[pl.ds(wid * 16, 16)], cbuf)
    n_in = cbuf[pl.ds(0, 16)][0]
    nch = n_in // CH

    lane = lax.iota(jnp.int32, 16)

    pltpu.sync_copy(crow_in.at[pl.ds(wid * CAP, CH)], idxr)
    pltpu.sync_copy(ccol_in.at[pl.ds(wid * CAP, CH)], idxc)
    pltpu.async_copy(hn.at[idxr], rr, semi)
    pltpu.async_copy(hn.at[idxc], rc, semi)

    def chunk(k, pos):
        pltpu.make_async_copy(hn.at[idxr], rr, semi).wait()
        pltpu.make_async_copy(hn.at[idxc], rc, semi).wait()

        for q in range(CH // 16):
            sidxr[pl.ds(q * 16, 16)] = idxr[pl.ds(q * 16, 16)]
            sidxc[pl.ds(q * 16, 16)] = idxc[pl.ds(q * 16, 16)]
        koff = pl.multiple_of(jnp.minimum(k + 1, nch - 1) * CH, 16)
        d1 = pltpu.async_copy(crow_in.at[pl.ds(wid * CAP + koff, CH)], idxr, semx)
        d2 = pltpu.async_copy(ccol_in.at[pl.ds(wid * CAP + koff, CH)], idxc, semx)

        def grp(g, pos):
            sv = jnp.zeros((16,), _f32)
            for e2 in range(16):
                e = g * 16 + e2
                acc = rr[e, pl.ds(0, 16)] * rc[e, pl.ds(0, 16)]
                for j in range(1, 8):
                    acc = acc + rr[e, pl.ds(16 * j, 16)] * rc[e, pl.ds(16 * j, 16)]
                sm = jnp.sum(acc)
                sv = jnp.where(lane == e2, sm, sv)
            m = sv >= TH
            kf = jnp.where(m, 1.0, 0.0)
            keepv[pl.ds(g * 16, 16)] = kf
            skv[pl.ds(g * 16, 16)] = sv * kf
            ir = sidxr[pl.ds(g * 16, 16)]
            ic = sidxc[pl.ds(g * 16, 16)]
            plsc.store_compressed(crow_v.at[pl.ds(pos, 16)], ir, mask=m)
            plsc.store_compressed(ccol_v.at[pl.ds(pos, 16)], ic, mask=m)
            plsc.store_compressed(csims_v.at[pl.ds(pos, 16)], sv, mask=m)
            npop = plsc.all_reduce_population_count(m)[0]
            return pos + npop

        pos = lax.fori_loop(0, CH // 16, grp, pos)
        d1.wait()
        d2.wait()
        pltpu.async_copy(hn.at[idxr], rr, semi)
        pltpu.async_copy(hn.at[idxc], rc, semi)
        pltpu.sync_copy(skv, degs.at[sidxr], add=True)
        pltpu.sync_copy(keepv, cnts.at[sidxc], add=True)
        return pos

    pos = lax.fori_loop(0, nch, chunk, jnp.int32(0))
    pltpu.make_async_copy(hn.at[idxr], rr, semi).wait()
    pltpu.make_async_copy(hn.at[idxc], rc, semi).wait()

    dumv = jnp.full((16,), DUM, jnp.int32)
    for q in range(CH // 16):
        crow_v[pl.ds(pos + q * 16, 16)] = dumv
        ccol_v[pl.ds(pos + q * 16, 16)] = dumv
        csims_v[pl.ds(pos + q * 16, 16)] = zz
    padded = jnp.maximum((pos + CH - 1) // CH, 1) * CH
    cbuf[pl.ds(0, 16)] = jnp.full((16,), padded, jnp.int32)

    pltpu.sync_copy(cbuf, counts_o.at[pl.ds(wid * 16, 16)])
    pltpu.sync_copy(crow_v, crow_o.at[pl.ds(wid * CAP, CAP)])
    pltpu.sync_copy(ccol_v, ccol_o.at[pl.ds(wid * CAP, CAP)])
    pltpu.sync_copy(csims_v, csims_o.at[pl.ds(wid * CAP, CAP)])

    plsc.subcore_barrier()
    pltpu.sync_copy(degs.at[pl.ds(sid * SEG, SEG)], zbuf)
    pltpu.sync_copy(zbuf, deg_o.at[cid, pl.ds(sid * SEG, SEG)])
    pltpu.sync_copy(cnts.at[pl.ds(sid * SEG, SEG)], zbuf)
    pltpu.sync_copy(zbuf, cnt_o.at[cid, pl.ds(sid * SEG, SEG)])


_pass_a = functools.partial(
    pl.kernel,
    _pass_a_body,
    out_type=(
        jax.ShapeDtypeStruct((NTILE * CAP,), jnp.int32),
        jax.ShapeDtypeStruct((NTILE * CAP,), jnp.int32),
        jax.ShapeDtypeStruct((NTILE * CAP,), _f32),
        jax.ShapeDtypeStruct((NTILE * 16,), jnp.int32),
        jax.ShapeDtypeStruct((NCORE, NPAD), _f32),
        jax.ShapeDtypeStruct((NCORE, NPAD), _f32),
    ),
    mesh=_mesh,
    scratch_types=[
        pltpu.VMEM((CH,), jnp.int32),
        pltpu.VMEM((CH,), jnp.int32),
        pltpu.VMEM((CH,), jnp.int32),
        pltpu.VMEM((CH,), jnp.int32),
        pltpu.VMEM((CH, D), _f32),
        pltpu.VMEM((CH, D), _f32),
        pltpu.VMEM((CH,), _f32),
        pltpu.VMEM((CH,), _f32),
        pltpu.VMEM((SEG,), _f32),
        pltpu.VMEM((16,), jnp.int32),
        pltpu.VMEM((CAP,), jnp.int32),
        pltpu.VMEM((CAP,), jnp.int32),
        pltpu.VMEM((CAP,), _f32),
        pltpu.VMEM_SHARED((NPAD,), _f32),
        pltpu.VMEM_SHARED((NPAD,), _f32),
        pltpu.SemaphoreType.DMA,
        pltpu.SemaphoreType.DMA,
    ],
    compiler_params=pltpu.CompilerParams(needs_layout_passes=False),
)()


def _pass_b_body(hw, crow, ccol, csims, counts, dis,
                 outp_o,
                 idxr, idxc, simv, cbuf, rows, disv, zrows,
                 outacc, sem):
    cid = lax.axis_index("c")
    sid = lax.axis_index("s")
    wid = cid * NSUB + sid

    pltpu.sync_copy(dis, disv)

    zz = jnp.zeros((16,), _f32)

    def zb(t, _):
        i = t // 8
        j = t % 8
        zrows[i, pl.ds(16 * j, 16)] = zz
        return 0

    lax.fori_loop(0, 160 * 8, zb, 0)
    for q in range(4):
        pltpu.sync_copy(zrows, outacc.at[pl.ds(sid * SEG + q * 160, 160), :])
    plsc.subcore_barrier()

    pltpu.sync_copy(counts.at[pl.ds(wid * 16, 16)], cbuf)
    n_in = cbuf[pl.ds(0, 16)][0]
    nch = n_in // CH

    def chunk(k, _):
        off = pl.multiple_of(k * CH, 16)
        pltpu.sync_copy(crow.at[pl.ds(wid * CAP + off, CH)], idxr)
        pltpu.sync_copy(ccol.at[pl.ds(wid * CAP + off, CH)], idxc)
        pltpu.sync_copy(csims.at[pl.ds(wid * CAP + off, CH)], simv)
        pltpu.async_copy(hw.at[idxc], rows, sem).wait()

        def grp(g, _):
            sl = simv[pl.ds(g * 16, 16)]
            ir = idxr[pl.ds(g * 16, 16)]
            ic = idxc[pl.ds(g * 16, 16)]
            dr = plsc.load_gather(disv, [ir])
            dc = plsc.load_gather(disv, [ic])
            vv = jnp.exp(dr * sl * dc)
            for e2 in range(16):
                e = g * 16 + e2
                v = vv[e2]
                for j in range(8):
                    rows[e, pl.ds(16 * j, 16)] = rows[e, pl.ds(16 * j, 16)] * v
            return 0

        lax.fori_loop(0, CH // 16, grp, 0)
        pltpu.sync_copy(rows, outacc.at[idxr], add=True)
        return 0

    lax.fori_loop(0, nch, chunk, 0)
    plsc.subcore_barrier()

    for q in range(4):
        pltpu.sync_copy(outacc.at[pl.ds(sid * SEG + q * 160, 160), :], zrows)
        pltpu.sync_copy(zrows, outp_o.at[cid, pl.ds(sid * SEG + q * 160, 160), :])


_pass_b = functools.partial(
    pl.kernel,
    _pass_b_body,
    out_type=jax.ShapeDtypeStruct((NCORE, NPAD, D), _f32),
    mesh=_mesh,
    scratch_types=[
        pltpu.VMEM((CH,), jnp.int32),
        pltpu.VMEM((CH,), jnp.int32),
        pltpu.VMEM((CH,), _f32),
        pltpu.VMEM((16,), jnp.int32),
        pltpu.VMEM((CH, D), _f32),
        pltpu.VMEM((NPAD,), _f32),
        pltpu.VMEM((160, D), _f32),
        pltpu.VMEM_SHARED((NPAD, D), _f32),
        pltpu.SemaphoreType.DMA,
    ],
    compiler_params=pltpu.CompilerParams(needs_layout_passes=False),
)()


def kernel(x, edge_index, W0, b0, W1, b1, g1, be1, g2, be2):
    row = edge_index[0].astype(jnp.int32)
    col = edge_index[1].astype(jnp.int32)
    h = jnp.zeros((NPAD, D), _f32).at[:N].set(x)

    crow = jnp.full((NTILE, CAP), DUM, jnp.int32).at[:, :EPT].set(
        row.reshape(NTILE, EPT)).reshape(NTILE * CAP)
    ccol = jnp.full((NTILE, CAP), DUM, jnp.int32).at[:, :EPT].set(
        col.reshape(NTILE, EPT)).reshape(NTILE * CAP)
    counts = jnp.full((NTILE * 16,), EPT, jnp.int32)

    params = [(W0, b0, g1, be1, 1.0, False),
              (W1, b1, g2, be2, 2.0, False),
              (W1, b1, None, None, 2.0, True)]

    for W, b, g, be, cadd, final in params:
        hn, hw = _dense_pre(h, W)
        crow, ccol, csims, counts, degp, cntp = _pass_a(hn, crow, ccol, counts)
        dis, sv = _norm_tc(degp, cntp, cadd)
        outp = _pass_b(hw, crow, ccol, csims, counts, dis)
        if final:
            h = _dense_final(outp, sv, hw, b)
        else:
            h = _dense_post(outp, sv, hw, b, g, be)

    return h[:N]


# speedup vs baseline: 1.0273x; 1.0273x over previous
import functools

import jax
import jax.numpy as jnp
from jax import lax
from jax.experimental import pallas as pl
from jax.experimental.pallas import tpu as pltpu
from jax.experimental.pallas import tpu_sc as plsc

N = 10000
E = 320000
D = 128
TH = 0.1
LN_EPS = 1e-5

NPAD = 10240
NCORE = 2
NSUB = 16
NTILE = NCORE * NSUB
EPT = E // NTILE
CH = 80
SEG = NPAD // NSUB

_f32 = jnp.float32

_mesh = plsc.VectorSubcoreMesh(core_axis_name="c", subcore_axis_name="s")


def _pre_body(h_ref, w_ref, hn_ref, hw_ref):
    h = h_ref[...]
    hw_ref[...] = jnp.dot(h, w_ref[...], preferred_element_type=_f32)
    nr = jnp.sqrt(jnp.sum(h * h, axis=1, keepdims=True))
    hn_ref[...] = h / jnp.maximum(nr, 1e-8)


def _dense_pre(h, W):
    return pl.pallas_call(
        _pre_body,
        grid=(8,),
        in_specs=[
            pl.BlockSpec((NPAD // 8, D), lambda i: (i, 0)),
            pl.BlockSpec((D, D), lambda i: (0, 0)),
        ],
        out_specs=[
            pl.BlockSpec((NPAD // 8, D), lambda i: (i, 0)),
            pl.BlockSpec((NPAD // 8, D), lambda i: (i, 0)),
        ],
        out_shape=[
            jax.ShapeDtypeStruct((NPAD, D), _f32),
            jax.ShapeDtypeStruct((NPAD, D), _f32),
        ],
    )(h, W)


def _norm_body(degp_ref, cntp_ref, dis_ref, sv_ref, *, cadd):
    deg = degp_ref[0] + degp_ref[1]
    dis_ref[...] = jnp.where(deg > 0.0, lax.rsqrt(deg), 0.0)
    cnt = cntp_ref[0] + cntp_ref[1]
    sv_ref[...] = jnp.exp(1.0 / (cnt + cadd))


def _norm_tc(degp, cntp, cadd):
    degp2 = degp.reshape(2, NPAD // 128, 128)
    cntp2 = cntp.reshape(2, NPAD // 128, 128)
    dis, sv = pl.pallas_call(
        functools.partial(_norm_body, cadd=cadd),
        out_shape=[
            jax.ShapeDtypeStruct((NPAD // 128, 128), _f32),
            jax.ShapeDtypeStruct((NPAD // 128, 128), _f32),
        ],
    )(degp2, cntp2)
    return dis.reshape(NPAD), sv.reshape(NPAD)


def _post_body(outp_ref, sv_ref, hw_ref, b_ref, g_ref, be_ref, h_ref):
    o = outp_ref[0] + outp_ref[1] + sv_ref[...] * hw_ref[...] + b_ref[...]
    m = jnp.mean(o, axis=1, keepdims=True)
    v = jnp.mean((o - m) ** 2, axis=1, keepdims=True)
    h_ref[...] = jnp.maximum(
        (o - m) / jnp.sqrt(v + LN_EPS) * g_ref[...] + be_ref[...], 0.0
    )


def _final_body(outp_ref, sv_ref, hw_ref, b_ref, h_ref):
    o = outp_ref[0] + outp_ref[1] + sv_ref[...] * hw_ref[...] + b_ref[...]
    z = o - jnp.max(o, axis=1, keepdims=True)
    h_ref[...] = z - jnp.log(jnp.sum(jnp.exp(z), axis=1, keepdims=True))


def _dense_post(outp, sv, hw, b, g, be):
    blk = NPAD // 8
    return pl.pallas_call(
        _post_body,
        grid=(8,),
        in_specs=[
            pl.BlockSpec((2, blk, D), lambda i: (0, i, 0)),
            pl.BlockSpec((blk, 1), lambda i: (i, 0)),
            pl.BlockSpec((blk, D), lambda i: (i, 0)),
            pl.BlockSpec((1, D), lambda i: (0, 0)),
            pl.BlockSpec((1, D), lambda i: (0, 0)),
            pl.BlockSpec((1, D), lambda i: (0, 0)),
        ],
        out_specs=pl.BlockSpec((blk, D), lambda i: (i, 0)),
        out_shape=jax.ShapeDtypeStruct((NPAD, D), _f32),
    )(outp, sv[:, None], hw, b[None, :], g[None, :], be[None, :])


def _dense_final(outp, sv, hw, b):
    blk = NPAD // 8
    return pl.pallas_call(
        _final_body,
        grid=(8,),
        in_specs=[
            pl.BlockSpec((2, blk, D), lambda i: (0, i, 0)),
            pl.BlockSpec((blk, 1), lambda i: (i, 0)),
            pl.BlockSpec((blk, D), lambda i: (i, 0)),
            pl.BlockSpec((1, D), lambda i: (0, 0)),
        ],
        out_specs=pl.BlockSpec((blk, D), lambda i: (i, 0)),
        out_shape=jax.ShapeDtypeStruct((NPAD, D), _f32),
    )(outp, sv[:, None], hw, b[None, :])


CAP = EPT + 2 * CH
DUM = NPAD - 1


def _pass_a_body(hn, crow_in, ccol_in, counts_in,
                 crow_o, ccol_o, csims_o, counts_o, deg_o, cnt_o,
                 idxr, idxc, rr, rc, keepv, skv, zbuf, cbuf,
                 crow_v, ccol_v, csims_v,
                 degs, cnts, semi):
    cid = lax.axis_index("c")
    sid = lax.axis_index("s")
    wid = cid * NSUB + sid

    zz = jnp.zeros((16,), _f32)

    def zb(i, _):
        zbuf[pl.ds(i * 16, 16)] = zz
        return 0

    lax.fori_loop(0, SEG // 16, zb, 0)
    pltpu.sync_copy(zbuf, degs.at[pl.ds(sid * SEG, SEG)])
    pltpu.sync_copy(zbuf, cnts.at[pl.ds(sid * SEG, SEG)])
    plsc.subcore_barrier()

    pltpu.sync_copy(counts_in.at[pl.ds(wid * 16, 16)], cbuf)
    n_in = cbuf[pl.ds(0, 16)][0]
    nch = n_in // CH

    lane = lax.iota(jnp.int32, 16)

    def chunk(k, pos):
        off = pl.multiple_of(k * CH, 16)
        pltpu.sync_copy(crow_in.at[pl.ds(wid * CAP + off, CH)], idxr)
        pltpu.sync_copy(ccol_in.at[pl.ds(wid * CAP + off, CH)], idxc)
        g1 = pltpu.async_copy(hn.at[idxr], rr, semi)
        g2 = pltpu.async_copy(hn.at[idxc], rc, semi)
        g1.wait()
        g2.wait()

        def grp(g, pos):
            sv = jnp.zeros((16,), _f32)
            for e2 in range(16):
                e = g * 16 + e2
                acc = rr[e, pl.ds(0, 16)] * rc[e, pl.ds(0, 16)]
                for j in range(1, 8):
                    acc = acc + rr[e, pl.ds(16 * j, 16)] * rc[e, pl.ds(16 * j, 16)]
                sm = jnp.sum(acc)
                sv = jnp.where(lane == e2, sm, sv)
            m = sv >= TH
            kf = jnp.where(m, 1.0, 0.0)
            keepv[pl.ds(g * 16, 16)] = kf
            skv[pl.ds(g * 16, 16)] = sv * kf
            ir = idxr[pl.ds(g * 16, 16)]
            ic = idxc[pl.ds(g * 16, 16)]
            plsc.store_compressed(crow_v.at[pl.ds(pos, 16)], ir, mask=m)
            plsc.store_compressed(ccol_v.at[pl.ds(pos, 16)], ic, mask=m)
            plsc.store_compressed(csims_v.at[pl.ds(pos, 16)], sv, mask=m)
            npop = plsc.all_reduce_population_count(m)[0]
            return pos + npop

        pos = lax.fori_loop(0, CH // 16, grp, pos)
        pltpu.sync_copy(skv, degs.at[idxr], add=True)
        pltpu.sync_copy(keepv, cnts.at[idxc], add=True)
        return pos

    pos = lax.fori_loop(0, nch, chunk, jnp.int32(0))

    dumv = jnp.full((16,), DUM, jnp.int32)
    for q in range(CH // 16):
        crow_v[pl.ds(pos + q * 16, 16)] = dumv
        ccol_v[pl.ds(pos + q * 16, 16)] = dumv
        csims_v[pl.ds(pos + q * 16, 16)] = zz
    padded = jnp.maximum((pos + CH - 1) // CH, 1) * CH
    cbuf[pl.ds(0, 16)] = jnp.full((16,), padded, jnp.int32)

    pltpu.sync_copy(cbuf, counts_o.at[pl.ds(wid * 16, 16)])
    pltpu.sync_copy(crow_v, crow_o.at[pl.ds(wid * CAP, CAP)])
    pltpu.sync_copy(ccol_v, ccol_o.at[pl.ds(wid * CAP, CAP)])
    pltpu.sync_copy(csims_v, csims_o.at[pl.ds(wid * CAP, CAP)])

    plsc.subcore_barrier()
    pltpu.sync_copy(degs.at[pl.ds(sid * SEG, SEG)], zbuf)
    pltpu.sync_copy(zbuf, deg_o.at[cid, pl.ds(sid * SEG, SEG)])
    pltpu.sync_copy(cnts.at[pl.ds(sid * SEG, SEG)], zbuf)
    pltpu.sync_copy(zbuf, cnt_o.at[cid, pl.ds(sid * SEG, SEG)])


_pass_a = functools.partial(
    pl.kernel,
    _pass_a_body,
    out_type=(
        jax.ShapeDtypeStruct((NTILE * CAP,), jnp.int32),
        jax.ShapeDtypeStruct((NTILE * CAP,), jnp.int32),
        jax.ShapeDtypeStruct((NTILE * CAP,), _f32),
        jax.ShapeDtypeStruct((NTILE * 16,), jnp.int32),
        jax.ShapeDtypeStruct((NCORE, NPAD), _f32),
        jax.ShapeDtypeStruct((NCORE, NPAD), _f32),
    ),
    mesh=_mesh,
    scratch_types=[
        pltpu.VMEM((CH,), jnp.int32),
        pltpu.VMEM((CH,), jnp.int32),
        pltpu.VMEM((CH, D), _f32),
        pltpu.VMEM((CH, D), _f32),
        pltpu.VMEM((CH,), _f32),
        pltpu.VMEM((CH,), _f32),
        pltpu.VMEM((SEG,), _f32),
        pltpu.VMEM((16,), jnp.int32),
        pltpu.VMEM((CAP,), jnp.int32),
        pltpu.VMEM((CAP,), jnp.int32),
        pltpu.VMEM((CAP,), _f32),
        pltpu.VMEM_SHARED((NPAD,), _f32),
        pltpu.VMEM_SHARED((NPAD,), _f32),
        pltpu.SemaphoreType.DMA,
    ],
    compiler_params=pltpu.CompilerParams(needs_layout_passes=False),
)()


def _pass_b_body(hw, crow, ccol, csims, counts, dis,
                 outp_o,
                 idxr, idxc, simv, cbuf, rows, disv, zrows,
                 outacc, sem):
    cid = lax.axis_index("c")
    sid = lax.axis_index("s")
    wid = cid * NSUB + sid

    pltpu.sync_copy(dis, disv)

    zz = jnp.zeros((16,), _f32)

    def zb(t, _):
        i = t // 8
        j = t % 8
        zrows[i, pl.ds(16 * j, 16)] = zz
        return 0

    lax.fori_loop(0, 160 * 8, zb, 0)
    for q in range(4):
        pltpu.sync_copy(zrows, outacc.at[pl.ds(sid * SEG + q * 160, 160), :])
    plsc.subcore_barrier()

    pltpu.sync_copy(counts.at[pl.ds(wid * 16, 16)], cbuf)
    n_in = cbuf[pl.ds(0, 16)][0]
    nch = n_in // CH

    def chunk(k, _):
        off = pl.multiple_of(k * CH, 16)
        pltpu.sync_copy(crow.at[pl.ds(wid * CAP + off, CH)], idxr)
        pltpu.sync_copy(ccol.at[pl.ds(wid * CAP + off, CH)], idxc)
        pltpu.sync_copy(csims.at[pl.ds(wid * CAP + off, CH)], simv)
        pltpu.async_copy(hw.at[idxc], rows, sem).wait()

        def grp(g, _):
            sl = simv[pl.ds(g * 16, 16)]
            ir = idxr[pl.ds(g * 16, 16)]
            ic = idxc[pl.ds(g * 16, 16)]
            dr = plsc.load_gather(disv, [ir])
            dc = plsc.load_gather(disv, [ic])
            vv = jnp.exp(dr * sl * dc)
            for e2 in range(16):
                e = g * 16 + e2
                v = vv[e2]
                for j in range(8):
                    rows[e, pl.ds(16 * j, 16)] = rows[e, pl.ds(16 * j, 16)] * v
            return 0

        lax.fori_loop(0, CH // 16, grp, 0)
        pltpu.sync_copy(rows, outacc.at[idxr], add=True)
        return 0

    lax.fori_loop(0, nch, chunk, 0)
    plsc.subcore_barrier()

    for q in range(4):
        pltpu.sync_copy(outacc.at[pl.ds(sid * SEG + q * 160, 160), :], zrows)
        pltpu.sync_copy(zrows, outp_o.at[cid, pl.ds(sid * SEG + q * 160, 160), :])


_pass_b = functools.partial(
    pl.kernel,
    _pass_b_body,
    out_type=jax.ShapeDtypeStruct((NCORE, NPAD, D), _f32),
    mesh=_mesh,
    scratch_types=[
        pltpu.VMEM((CH,), jnp.int32),
        pltpu.VMEM((CH,), jnp.int32),
        pltpu.VMEM((CH,), _f32),
        pltpu.VMEM((16,), jnp.int32),
        pltpu.VMEM((CH, D), _f32),
        pltpu.VMEM((NPAD,), _f32),
        pltpu.VMEM((160, D), _f32),
        pltpu.VMEM_SHARED((NPAD, D), _f32),
        pltpu.SemaphoreType.DMA,
    ],
    compiler_params=pltpu.CompilerParams(needs_layout_passes=False),
)()


def kernel(x, edge_index, W0, b0, W1, b1, g1, be1, g2, be2):
    row = edge_index[0].astype(jnp.int32)
    col = edge_index[1].astype(jnp.int32)
    h = jnp.zeros((NPAD, D), _f32).at[:N].set(x)

    crow = jnp.full((NTILE, CAP), DUM, jnp.int32).at[:, :EPT].set(
        row.reshape(NTILE, EPT)).reshape(NTILE * CAP)
    ccol = jnp.full((NTILE, CAP), DUM, jnp.int32).at[:, :EPT].set(
        col.reshape(NTILE, EPT)).reshape(NTILE * CAP)
    counts = jnp.full((NTILE * 16,), EPT, jnp.int32)

    params = [(W0, b0, g1, be1, 1.0, False),
              (W1, b1, g2, be2, 2.0, False),
              (W1, b1, None, None, 2.0, True)]

    for W, b, g, be, cadd, final in params:
        hn, hw = _dense_pre(h, W)
        crow, ccol, csims, counts, degp, cntp = _pass_a(hn, crow, ccol, counts)
        dis, sv = _norm_tc(degp, cntp, cadd)
        outp = _pass_b(hw, crow, ccol, csims, counts, dis)
        if final:
            h = _dense_final(outp, sv, hw, b)
        else:
            h = _dense_post(outp, sv, hw, b, g, be)

    return h[:N]
